# baseline (speedup 1.0000x reference)
import jax
import jax.numpy as jnp
from jax.experimental import pallas as pl
from jax.experimental.pallas import tpu as pltpu

EPS = 1e-5
NEG_SLOPE = 0.01
LANE = 128


def _bn_train(x, gamma, beta):
    mu = jnp.mean(x, axis=0, keepdims=True)
    var = jnp.mean((x - mu) * (x - mu), axis=0, keepdims=True)
    return (x - mu) * jax.lax.rsqrt(var + EPS) * gamma + beta


def _leaky_relu(x):
    return jnp.where(x > 0, x, NEG_SLOPE * x)


def _choose_hw_tile(HW, N, C, itemsize, budget_bytes=8 << 20):
    best = None
    cap = max(1, budget_bytes // (N * C * itemsize))
    for t in range(1, HW + 1):
        if HW % t == 0 and t % 8 == 0 and t <= cap:
            best = t
    if best is None:
        for t in range(1, HW + 1):
            if HW % t == 0 and t <= cap:
                best = t
    return best or HW


def _make_fused_kernel(grid_k, n_streams, inv_hw):
    last_k = grid_k - 1

    def _kernel_body(*refs):
        x_refs = refs[:n_streams]
        (w1_ref, b1_ref, g1_ref, be1_ref,
         w2_ref, b2_ref, g2_ref, be2_ref,
         w3_ref, b3_ref, g3_ref, be3_ref,
         w4_ref, b4_ref, wm_ref, bm_ref,
         wd_ref, bd_ref, wc_ref, bc_ref,
         set_ref, cls_ref, mix_ref,
         acc_ref) = refs[n_streams:]
        k = pl.program_id(0)
        partial = jnp.sum(x_refs[0][...], axis=1)
        for r in x_refs[1:]:
            partial += jnp.sum(r[...], axis=1)

        if grid_k > 1:
            @pl.when(k == 0)
            def _():
                acc_ref[...] = partial

            @pl.when(k > 0)
            def _():
                acc_ref[...] += partial

        @pl.when(k == last_k)
        def _():
            if grid_k > 1:
                feats = acc_ref[...] * inv_hw
            else:
                feats = partial * inv_hw

            h = (jnp.dot(feats, w1_ref[...], preferred_element_type=jnp.float32)
                 + b1_ref[...])
            h = _leaky_relu(_bn_train(h, g1_ref[...], be1_ref[...]))
            h = (jnp.dot(h, w2_ref[...], preferred_element_type=jnp.float32)
                 + b2_ref[...])
            h = _leaky_relu(_bn_train(h, g2_ref[...], be2_ref[...]))
            h = (jnp.dot(h, w3_ref[...], preferred_element_type=jnp.float32)
                 + b3_ref[...])
            h = _leaky_relu(_bn_train(h, g3_ref[...], be3_ref[...]))

            h4 = (jnp.dot(h, w4_ref[...], preferred_element_type=jnp.float32)
                  + b4_ref[...])
            mix = jax.nn.sigmoid(
                jnp.dot(h4, wm_ref[...], preferred_element_type=jnp.float32)
                + bm_ref[...])

            set_info = feats * mix
            class_info = feats - set_info
            set_ref[...] = (
                jnp.dot(set_info, wd_ref[...], preferred_element_type=jnp.float32)
                + bd_ref[...])
            cls_ref[...] = (
                jnp.dot(class_info, wc_ref[...], preferred_element_type=jnp.float32)
                + bc_ref[...])
            mix_ref[...] = mix

    return _kernel_body


def kernel(feats, w1, b1, g1, be1, w2, b2, g2, be2, w3, b3, g3, be3,
           w4, b4, wm, bm, wd, bd, wc, bc):
    N, C, H, W = feats.shape
    K = wc.shape[1]
    HW = H * W
    itemsize = jnp.dtype(feats.dtype).itemsize

    xt = jnp.transpose(feats, (0, 2, 3, 1)).reshape(N, HW, C)

    hw_tile = _choose_hw_tile(HW, N, C, itemsize)
    n_blocks = HW // hw_tile
    n_streams = 2 if n_blocks % 2 == 0 else 1
    grid_k = n_blocks // n_streams

    def _whole(a):
        nd = a.ndim
        return pl.BlockSpec(a.shape, lambda k, _n=nd: (0,) * _n)

    weights = (w1, b1, g1, be1, w2, b2, g2, be2, w3, b3, g3, be3,
               w4, b4, wm, bm, wd, bd, wc, bc)

    x_specs = [
        pl.BlockSpec((N, hw_tile, C), lambda k, _j=j: (0, k * n_streams + _j, 0))
        for j in range(n_streams)
    ]

    set_preds, class_preds, mix_factor = pl.pallas_call(
        _make_fused_kernel(grid_k, n_streams, 1.0 / float(HW)),
        out_shape=(
            jax.ShapeDtypeStruct((N, 1), jnp.float32),
            jax.ShapeDtypeStruct((N, K), jnp.float32),
            jax.ShapeDtypeStruct((N, C), jnp.float32),
        ),
        grid=(grid_k,),
        in_specs=x_specs + [_whole(a) for a in weights],
        out_specs=(
            pl.BlockSpec((N, 1), lambda k: (0, 0)),
            pl.BlockSpec((N, K), lambda k: (0, 0)),
            pl.BlockSpec((N, C), lambda k: (0, 0)),
        ),
        scratch_shapes=[pltpu.VMEM((N, C), jnp.float32)],
        compiler_params=pltpu.CompilerParams(
            dimension_semantics=("arbitrary",),
            vmem_limit_bytes=96 << 20,
        ),
        cost_estimate=pl.CostEstimate(
            flops=int(N * C * HW),
            transcendentals=int(N * C),
            bytes_accessed=int(N * C * HW * itemsize),
        ),
    )(*([xt] * n_streams), *weights)

    return set_preds, class_preds, mix_factor


# speedup vs baseline: 1.0207x; 1.0207x over previous
import jax
import jax.numpy as jnp
from jax.experimental import pallas as pl
from jax.experimental.pallas import tpu as pltpu

EPS = 1e-5
NEG_SLOPE = 0.01
LANE = 128


def _bn_train(x, gamma, beta):
    mu = jnp.mean(x, axis=0, keepdims=True)
    var = jnp.mean((x - mu) * (x - mu), axis=0, keepdims=True)
    return (x - mu) * jax.lax.rsqrt(var + EPS) * gamma + beta


def _leaky_relu(x):
    return jnp.where(x > 0, x, NEG_SLOPE * x)


def _choose_hw_tile(HW, N, C, itemsize, budget_bytes=8 << 20):
    best = None
    cap = max(1, budget_bytes // (N * C * itemsize))
    for t in range(1, HW + 1):
        if HW % t == 0 and t % 8 == 0 and t <= cap:
            best = t
    if best is None:
        for t in range(1, HW + 1):
            if HW % t == 0 and t <= cap:
                best = t
    return best or HW


def _make_fused_kernel(grid_k, n_streams, inv_hw):
    last_k = grid_k - 1

    def _kernel_body(*refs):
        x_refs = refs[:n_streams]
        (w1_ref, b1_ref, g1_ref, be1_ref,
         w2_ref, b2_ref, g2_ref, be2_ref,
         w3_ref, b3_ref, g3_ref, be3_ref,
         w4_ref, b4_ref, wm_ref, bm_ref,
         wd_ref, bd_ref, wc_ref, bc_ref,
         set_ref, cls_ref, mix_ref,
         acc_ref) = refs[n_streams:]
        k = pl.program_id(0)
        partial = jnp.sum(x_refs[0][...], axis=1)
        for r in x_refs[1:]:
            partial += jnp.sum(r[...], axis=1)

        if grid_k > 1:
            @pl.when(k == 0)
            def _():
                acc_ref[...] = partial

            @pl.when(k > 0)
            def _():
                acc_ref[...] += partial

        @pl.when(k == last_k)
        def _():
            if grid_k > 1:
                feats = acc_ref[...] * inv_hw
            else:
                feats = partial * inv_hw

            h = (jnp.dot(feats, w1_ref[...], preferred_element_type=jnp.float32)
                 + b1_ref[...])
            h = _leaky_relu(_bn_train(h, g1_ref[...], be1_ref[...]))
            h = (jnp.dot(h, w2_ref[...], preferred_element_type=jnp.float32)
                 + b2_ref[...])
            h = _leaky_relu(_bn_train(h, g2_ref[...], be2_ref[...]))
            h = (jnp.dot(h, w3_ref[...], preferred_element_type=jnp.float32)
                 + b3_ref[...])
            h = _leaky_relu(_bn_train(h, g3_ref[...], be3_ref[...]))

            h4 = (jnp.dot(h, w4_ref[...], preferred_element_type=jnp.float32)
                  + b4_ref[...])
            mix = jax.nn.sigmoid(
                jnp.dot(h4, wm_ref[...], preferred_element_type=jnp.float32)
                + bm_ref[...])

            set_info = feats * mix
            class_info = feats - set_info
            set_ref[...] = (
                jnp.dot(set_info, wd_ref[...], preferred_element_type=jnp.float32)
                + bd_ref[...])
            cls_ref[...] = (
                jnp.dot(class_info, wc_ref[...], preferred_element_type=jnp.float32)
                + bc_ref[...])
            mix_ref[...] = mix

    return _kernel_body


def kernel(feats, w1, b1, g1, be1, w2, b2, g2, be2, w3, b3, g3, be3,
           w4, b4, wm, bm, wd, bd, wc, bc):
    N, C, H, W = feats.shape
    K = wc.shape[1]
    HW = H * W
    itemsize = jnp.dtype(feats.dtype).itemsize

    xt = jnp.transpose(feats, (0, 2, 3, 1)).reshape(N, HW, C)

    hw_tile = _choose_hw_tile(HW, N, C, itemsize)
    n_streams = 1
    grid_k = HW // hw_tile

    def _whole(a):
        nd = a.ndim
        return pl.BlockSpec(a.shape, lambda k, _n=nd: (0,) * _n)

    weights = (w1, b1, g1, be1, w2, b2, g2, be2, w3, b3, g3, be3,
               w4, b4, wm, bm, wd, bd, wc, bc)

    x_specs = [
        pl.BlockSpec((N, hw_tile, C), lambda k, _j=j: (0, k * n_streams + _j, 0))
        for j in range(n_streams)
    ]

    set_preds, class_preds, mix_factor = pl.pallas_call(
        _make_fused_kernel(grid_k, n_streams, 1.0 / float(HW)),
        out_shape=(
            jax.ShapeDtypeStruct((N, 1), jnp.float32),
            jax.ShapeDtypeStruct((N, K), jnp.float32),
            jax.ShapeDtypeStruct((N, C), jnp.float32),
        ),
        grid=(grid_k,),
        in_specs=x_specs + [_whole(a) for a in weights],
        out_specs=(
            pl.BlockSpec((N, 1), lambda k: (0, 0)),
            pl.BlockSpec((N, K), lambda k: (0, 0)),
            pl.BlockSpec((N, C), lambda k: (0, 0)),
        ),
        scratch_shapes=[pltpu.VMEM((N, C), jnp.float32)],
        compiler_params=pltpu.CompilerParams(
            dimension_semantics=("arbitrary",),
            vmem_limit_bytes=96 << 20,
        ),
        cost_estimate=pl.CostEstimate(
            flops=int(N * C * HW),
            transcendentals=int(N * C),
            bytes_accessed=int(N * C * HW * itemsize),
        ),
    )(*([xt] * n_streams), *weights)

    return set_preds, class_preds, mix_factor
